# Initial kernel scaffold; baseline (speedup 1.0000x reference)
import jax
import jax.numpy as jnp
from jax.experimental import pallas as pl


def kernel(x, router_logits):
    raise NotImplementedError("write your pallas kernel here")


# speedup vs baseline: 1.3918x; 1.3918x over previous
import functools

import jax
import jax.numpy as jnp
from jax import lax
from jax.experimental import pallas as pl
from jax.experimental.pallas import tpu as pltpu
from jax.experimental.pallas import tpu_sc as plsc

_T = 4096
_D = 2048
_E = 64
_C = 256
_L = 16


def _routing_body(logits_ref, src_ref, wrep_ref, probs_ref, dst_ref):
    e = pl.program_id(0)

    @pl.when(e == 0)
    def _init():
        l = logits_ref[...]
        m = jnp.max(l, axis=1, keepdims=True)
        p = jnp.exp(l - m)
        probs = p / jnp.sum(p, axis=1, keepdims=True)
        probs_ref[...] = probs
        cols = lax.broadcasted_iota(jnp.int32, (_T, _E), 1)
        m1 = jnp.max(probs, axis=1, keepdims=True)
        i1 = jnp.min(jnp.where(probs == m1, cols, _E), axis=1, keepdims=True)
        p2 = jnp.where(cols == i1, -jnp.inf, probs)
        m2 = jnp.max(p2, axis=1, keepdims=True)
        i2 = jnp.min(jnp.where(p2 == m2, cols, _E), axis=1, keepdims=True)
        mask = ((cols == i1) | (cols == i2)).astype(jnp.int32)
        c = mask
        k = 1
        while k < _T:
            c = c + jnp.concatenate(
                [jnp.zeros((k, _E), jnp.int32), c[:-k]], axis=0)
            k *= 2
        dst_ref[...] = c * mask

    e_cols = lax.broadcasted_iota(jnp.int32, (_T, _E), 1)
    sel = e_cols == e
    dst_col = jnp.sum(jnp.where(sel, dst_ref[...], 0), axis=1)
    p_col = jnp.sum(jnp.where(sel, probs_ref[...], 0.0), axis=1)
    slot = lax.broadcasted_iota(jnp.int32, (_C, _T), 0) + 1
    eq = slot == dst_col[None, :]
    ids = lax.broadcasted_iota(jnp.int32, (_C, _T), 1)
    src_row = jnp.sum(jnp.where(eq, ids, 0), axis=1)
    w_row = jnp.sum(jnp.where(eq, p_col[None, :], 0.0), axis=1)
    src_ref[...] = src_row.reshape(1, 1, _C)
    wrep_ref[...] = jnp.broadcast_to(w_row[:, None], (_C, _L))


def _routing(logits):
    return pl.pallas_call(
        _routing_body,
        grid=(_E,),
        in_specs=[pl.BlockSpec((_T, _E), lambda e: (0, 0))],
        out_specs=[
            pl.BlockSpec((1, 1, _C), lambda e: (e, 0, 0)),
            pl.BlockSpec((_C, _L), lambda e: (e, 0)),
        ],
        out_shape=[
            jax.ShapeDtypeStruct((_E, 1, _C), jnp.int32),
            jax.ShapeDtypeStruct((_E * _C, _L), jnp.float32),
        ],
        scratch_shapes=[
            pltpu.VMEM((_T, _E), jnp.float32),
            pltpu.VMEM((_T, _E), jnp.int32),
        ],
    )(logits)


def _dispatch(x, src_flat, wrep):
    info = plsc.get_sparse_core_info()
    nc, ns = info.num_cores, info.num_subcores
    nw = nc * ns
    b = _E * _C
    bpw = b // nw
    r = _L
    nch = bpw // r

    @functools.partial(
        pl.kernel,
        mesh=plsc.VectorSubcoreMesh(core_axis_name="c", subcore_axis_name="s"),
        out_type=jax.ShapeDtypeStruct((b, _D), jnp.float32),
        scratch_types=[
            pltpu.VMEM((bpw,), jnp.int32),
            pltpu.VMEM((r, _L), jnp.float32),
            pltpu.VMEM((r, _D), jnp.float32),
            pltpu.SemaphoreType.DMA,
        ],
    )
    def k(x_hbm, src_hbm, wrep_hbm, out_hbm, idx_v, w_v, rows_v, sem):
        wid = lax.axis_index("s") * nc + lax.axis_index("c")
        base = wid * bpw
        pltpu.sync_copy(src_hbm.at[pl.ds(base, bpw)], idx_v)

        def chunk(g, carry):
            row0 = base + g * r
            idx_reg = idx_v[pl.ds(g * r, r)]
            pltpu.async_copy(x_hbm.at[idx_reg], rows_v, sem).wait()
            pltpu.sync_copy(wrep_hbm.at[pl.ds(row0, r)], w_v)

            def row(i, c2):
                wv = w_v[i]

                def col(cc, c3):
                    sl = pl.ds(cc * _L, _L)
                    rows_v[i, sl] = rows_v[i, sl] * wv
                    return c3

                return lax.fori_loop(0, _D // _L, col, c2)

            lax.fori_loop(0, r, row, 0)
            pltpu.sync_copy(rows_v, out_hbm.at[pl.ds(row0, r)])
            return carry

        lax.fori_loop(0, nch, chunk, 0)

    return k(x, src_flat, wrep)


def kernel(x, router_logits):
    src3, wrep = _routing(router_logits)
    src_flat = src3.reshape(_E * _C)
    out = _dispatch(x, src_flat, wrep)
    return out.reshape(_E, _C, _D)


# speedup vs baseline: 1.8171x; 1.3055x over previous
import functools

import jax
import jax.numpy as jnp
from jax import lax
from jax.experimental import pallas as pl
from jax.experimental.pallas import tpu as pltpu
from jax.experimental.pallas import tpu_sc as plsc

_T = 4096
_D = 2048
_E = 64
_C = 256
_L = 16


def _routing_body(logits_ref, src_ref, wrep_ref, probs_ref, dst_ref):
    e = pl.program_id(0)

    @pl.when(e == 0)
    def _init():
        l = logits_ref[...]
        m = jnp.max(l, axis=1, keepdims=True)
        p = jnp.exp(l - m)
        probs = p / jnp.sum(p, axis=1, keepdims=True)
        probs_ref[...] = probs
        cols = lax.broadcasted_iota(jnp.int32, (_T, _E), 1)
        m1 = jnp.max(probs, axis=1, keepdims=True)
        i1 = jnp.min(jnp.where(probs == m1, cols, _E), axis=1, keepdims=True)
        p2 = jnp.where(cols == i1, -jnp.inf, probs)
        m2 = jnp.max(p2, axis=1, keepdims=True)
        i2 = jnp.min(jnp.where(p2 == m2, cols, _E), axis=1, keepdims=True)
        mask = ((cols == i1) | (cols == i2)).astype(jnp.int32)
        c = mask
        k = 1
        while k < _T:
            c = c + jnp.concatenate(
                [jnp.zeros((k, _E), jnp.int32), c[:-k]], axis=0)
            k *= 2
        dst_ref[...] = c * mask

    e_cols = lax.broadcasted_iota(jnp.int32, (_T, _E), 1)
    sel = e_cols == e
    dst_col = jnp.sum(jnp.where(sel, dst_ref[...], 0), axis=1)
    p_col = jnp.sum(jnp.where(sel, probs_ref[...], 0.0), axis=1)
    slot = lax.broadcasted_iota(jnp.int32, (_C, _T), 0) + 1
    eq = slot == dst_col[None, :]
    ids = lax.broadcasted_iota(jnp.int32, (_C, _T), 1)
    src_row = jnp.sum(jnp.where(eq, ids, 0), axis=1)
    w_row = jnp.sum(jnp.where(eq, p_col[None, :], 0.0), axis=1)
    src_ref[...] = src_row.reshape(1, 1, _C)
    wrep_ref[...] = w_row.reshape(1, 1, _C)


def _routing(logits):
    return pl.pallas_call(
        _routing_body,
        grid=(_E,),
        in_specs=[pl.BlockSpec((_T, _E), lambda e: (0, 0))],
        out_specs=[
            pl.BlockSpec((1, 1, _C), lambda e: (e, 0, 0)),
            pl.BlockSpec((1, 1, _C), lambda e: (e, 0, 0)),
        ],
        out_shape=[
            jax.ShapeDtypeStruct((_E, 1, _C), jnp.int32),
            jax.ShapeDtypeStruct((_E, 1, _C), jnp.float32),
        ],
        scratch_shapes=[
            pltpu.VMEM((_T, _E), jnp.float32),
            pltpu.VMEM((_T, _E), jnp.int32),
        ],
    )(logits)


def _dispatch(x, src_flat, w_flat):
    info = plsc.get_sparse_core_info()
    nc, ns = info.num_cores, info.num_subcores
    nw = nc * ns
    b = _E * _C
    bpw = b // nw
    r = _L
    nch = bpw // r

    @functools.partial(
        pl.kernel,
        mesh=plsc.VectorSubcoreMesh(core_axis_name="c", subcore_axis_name="s"),
        out_type=jax.ShapeDtypeStruct((b, _D), jnp.float32),
        scratch_types=[
            pltpu.VMEM((bpw,), jnp.int32),
            pltpu.VMEM((bpw,), jnp.float32),
            pltpu.VMEM((r, _D), jnp.float32),
            pltpu.VMEM((r, _D), jnp.float32),
            pltpu.SemaphoreType.DMA,
            pltpu.SemaphoreType.DMA,
            pltpu.SemaphoreType.DMA,
            pltpu.SemaphoreType.DMA,
        ],
    )
    def k(x_hbm, src_hbm, w_hbm, out_hbm,
          idx_v, w_v, buf0, buf1, g0, g1, s0, s1):
        wid = lax.axis_index("s") * nc + lax.axis_index("c")
        base = wid * bpw
        pltpu.sync_copy(src_hbm.at[pl.ds(base, bpw)], idx_v)
        pltpu.sync_copy(w_hbm.at[pl.ds(base, bpw)], w_v)

        def start_gather(g, buf, sem):
            idx_reg = idx_v[pl.ds(g * r, r)]
            pltpu.async_copy(x_hbm.at[idx_reg], buf, sem)

        def wait_dma(buf, sem):
            pltpu.make_async_copy(x_hbm.at[pl.ds(0, r)], buf, sem).wait()

        def start_store(g, buf, sem):
            pltpu.async_copy(buf, out_hbm.at[pl.ds(base + g * r, r)], sem)

        def scale(g, buf):
            wgrp = w_v[pl.ds(g * r, r)]
            for i in range(r):
                ws = wgrp[i]

                def col(u, carry, i=i, ws=ws):
                    for v in range(16):
                        sl = pl.ds(u * (_L * 16) + v * _L, _L)
                        buf[i, sl] = buf[i, sl] * ws
                    return carry

                lax.fori_loop(0, _D // (_L * 16), col, 0)

        start_gather(0, buf0, g0)
        start_gather(1, buf1, g1)

        def step(kk, carry):
            ga = 2 * kk
            gb = 2 * kk + 1
            wait_dma(buf0, g0)
            scale(ga, buf0)
            start_store(ga, buf0, s0)
            wait_dma(buf1, g1)
            scale(gb, buf1)
            start_store(gb, buf1, s1)

            @pl.when(ga + 2 < nch)
            def _():
                wait_dma(buf0, s0)
                start_gather(ga + 2, buf0, g0)

            @pl.when(gb + 2 < nch)
            def _():
                wait_dma(buf1, s1)
                start_gather(gb + 2, buf1, g1)

            return carry

        lax.fori_loop(0, nch // 2, step, 0)
        wait_dma(buf0, s0)
        wait_dma(buf1, s1)

    return k(x, src_flat, w_flat)


def kernel(x, router_logits):
    src3, w3 = _routing(router_logits)
    src_flat = src3.reshape(_E * _C)
    w_flat = w3.reshape(_E * _C)
    out = _dispatch(x, src_flat, w_flat)
    return out.reshape(_E, _C, _D)
